# baseline (speedup 1.0000x reference)
import functools

import numpy as np
import jax
import jax.numpy as jnp
from jax import lax
from jax.experimental import pallas as pl
from jax.experimental.pallas import tpu as pltpu
from jax.experimental.pallas import tpu_sc as plsc

_N_LEVELS = 16
_LOG2_T = 19
_T = 1 << _LOG2_T
_TMASK = _T - 1
_BASE_RES = 16
_FINE_RES = 2048
_N = 262144
_SCALE = float(np.exp(np.log(_FINE_RES / _BASE_RES) / (_N_LEVELS - 1)))
_RES = [int(np.floor(_BASE_RES * _SCALE ** l)) for l in range(_N_LEVELS)]
_P2 = 2654435761
_P3 = 805459861

_NC, _NS, _LANES = 2, 16, 16
_NW = _NC * _NS
_C = 256
_G = _C // _LANES
_PW = _N // _NW
_CHUNKS = _PW // _C

_F = 2 * _N_LEVELS


def _sc_body(dirs_hbm, table_hbm, out_hbm, xs, ys, zs, idxb, wb, gb, feat, sem):
    wid = lax.axis_index("s") * _NC + lax.axis_index("c")
    iota = lax.iota(jnp.int32, _LANES)
    zero_i = jnp.zeros((_LANES,), jnp.int32)
    one_i = jnp.ones((_LANES,), jnp.int32)

    def chunk_body(ch, carry):
        base = wid * _PW + ch * _C
        pltpu.sync_copy(dirs_hbm.at[0, pl.ds(base, _C)], xs)
        pltpu.sync_copy(dirs_hbm.at[1, pl.ds(base, _C)], ys)
        pltpu.sync_copy(dirs_hbm.at[2, pl.ds(base, _C)], zs)

        for lvl in range(_N_LEVELS):
            res = float(_RES[lvl])
            lvl_off = lvl * _T

            def grp1(g, _, res=res, lvl_off=lvl_off):
                s = pl.ds(g * _LANES, _LANES)
                px = (xs[s] * 0.49 + 0.49) * res
                py = (ys[s] * 0.49 + 0.49) * res
                pz = (zs[s] * 0.49 + 0.49) * res
                xi = px.astype(jnp.int32)
                yi = py.astype(jnp.int32)
                zi = pz.astype(jnp.int32)
                wx = px - xi.astype(jnp.float32)
                wy = py - yi.astype(jnp.float32)
                wz = pz - zi.astype(jnp.float32)
                wx = wx * wx * (3.0 - 2.0 * wx)
                wy = wy * wy * (3.0 - 2.0 * wy)
                wz = wz * wz * (3.0 - 2.0 * wz)
                x0 = xi.astype(jnp.uint32)
                y0 = yi.astype(jnp.uint32)
                z0 = zi.astype(jnp.uint32)
                hx = [x0, x0 + jnp.uint32(1)]
                hy0 = y0 * jnp.uint32(_P2)
                hz0 = z0 * jnp.uint32(_P3)
                hy = [hy0, hy0 + jnp.uint32(_P2)]
                hz = [hz0, hz0 + jnp.uint32(_P3)]
                wx_ = [1.0 - wx, wx]
                syz = [(1.0 - wy) * (1.0 - wz), wy * (1.0 - wz),
                       (1.0 - wy) * wz, wy * wz]
                row = idxb.at[g]
                for c in range(8):
                    bx, by, bz = c & 1, (c >> 1) & 1, (c >> 2) & 1
                    h = hx[bx] ^ hy[by] ^ hz[bz]
                    idx = (h & jnp.uint32(_TMASK)).astype(jnp.int32) + lvl_off
                    row[pl.ds(c * _LANES, _LANES)] = idx
                    wb[c, s] = wx_[bx] * syz[by + 2 * bz]
                return _

            lax.fori_loop(0, _G, grp1, 0)

            cps = [pltpu.async_copy(table_hbm.at[idxb.at[g]], gb.at[g], sem)
                   for g in range(_G)]
            for cp in cps:
                cp.wait()

            def grp2(g, _, lvl=lvl):
                s = pl.ds(g * _LANES, _LANES)
                rows = gb.at[g]
                f0 = jnp.zeros((_LANES,), jnp.float32)
                f1 = jnp.zeros((_LANES,), jnp.float32)
                for c in range(8):
                    ridx = iota + c * _LANES
                    g0 = plsc.load_gather(rows, [ridx, zero_i])
                    g1 = plsc.load_gather(rows, [ridx, one_i])
                    wc = wb[c, s]
                    f0 = f0 + wc * g0
                    f1 = f1 + wc * g1
                feat[2 * lvl, s] = f0
                feat[2 * lvl + 1, s] = f1
                return _

            lax.fori_loop(0, _G, grp2, 0)

        pltpu.sync_copy(feat, out_hbm.at[:, pl.ds(base, _C)])
        return carry

    lax.fori_loop(0, _CHUNKS, chunk_body, 0)


@functools.lru_cache(maxsize=None)
def _build_sc_features():
    return pl.kernel(
        _sc_body,
        out_type=jax.ShapeDtypeStruct((_F, _N), jnp.float32),
        mesh=plsc.VectorSubcoreMesh(core_axis_name="c", subcore_axis_name="s",
                                    num_cores=_NC, num_subcores=_NS),
        scratch_types=[
            pltpu.VMEM((_C,), jnp.float32),
            pltpu.VMEM((_C,), jnp.float32),
            pltpu.VMEM((_C,), jnp.float32),
            pltpu.VMEM((_G, 8 * _LANES), jnp.int32),
            pltpu.VMEM((8, _C), jnp.float32),
            pltpu.VMEM((_G, 8 * _LANES, 2), jnp.float32),
            pltpu.VMEM((_F, _C), jnp.float32),
            pltpu.SemaphoreType.DMA,
        ],
        compiler_params=pltpu.CompilerParams(use_tc_tiling_on_sc=False,
                                             needs_layout_passes=False),
    )


_B = 2048


def _mlp_body(feat_ref, dirs_ref, w0f_ref, w0d_ref, b0_ref, w1_ref, b1_ref,
              w2_ref, b2_ref, out_ref):
    fb = feat_ref[...]
    db = dirs_ref[...]
    dn = (((0,), (0,)), ((), ()))
    h = lax.dot_general(fb, w0f_ref[...], dn, preferred_element_type=jnp.float32)
    h = h + lax.dot_general(db, w0d_ref[...], dn, preferred_element_type=jnp.float32)
    h = jnp.maximum(h + b0_ref[...], 0.0)
    h = jnp.maximum(
        jnp.dot(h, w1_ref[...], preferred_element_type=jnp.float32) + b1_ref[...],
        0.0)
    o = jnp.sum(h * w2_ref[...], axis=1) + b2_ref[0, 0] + 1.0
    dist = jnp.maximum(o, 0.0) + jnp.log(1.0 + jnp.exp(-jnp.abs(o)))
    out_ref[...] = dist


_mlp = pl.pallas_call(
    _mlp_body,
    grid=(_N // _B,),
    in_specs=[
        pl.BlockSpec((_F, _B), lambda i: (0, i)),
        pl.BlockSpec((3, _B), lambda i: (0, i)),
        pl.BlockSpec((_F, 64), lambda i: (0, 0)),
        pl.BlockSpec((3, 64), lambda i: (0, 0)),
        pl.BlockSpec((1, 64), lambda i: (0, 0)),
        pl.BlockSpec((64, 64), lambda i: (0, 0)),
        pl.BlockSpec((1, 64), lambda i: (0, 0)),
        pl.BlockSpec((1, 64), lambda i: (0, 0)),
        pl.BlockSpec(memory_space=pltpu.SMEM),
    ],
    out_specs=pl.BlockSpec((_B,), lambda i: (i,)),
    out_shape=jax.ShapeDtypeStruct((_N,), jnp.float32),
)


def kernel(directions, table, W0, b0, W1, b1, W2, b2):
    dirs_t = directions.T
    table_flat = table.reshape(_N_LEVELS * _T, 2)
    feat_t = _build_sc_features()(dirs_t, table_flat)
    return _mlp(feat_t, dirs_t, W0[3:], W0[:3], b0.reshape(1, 64), W1,
                b1.reshape(1, 64), W2.reshape(1, 64), b2.reshape(1, 1))
